# baseline (speedup 1.0000x reference)
import jax
import jax.numpy as jnp
from jax import lax
from jax.experimental import pallas as pl
from jax.experimental.pallas import tpu as pltpu
from jax.experimental.pallas import tpu_sc as plsc

NODE_SIZE = 1000000
EMBED_DIM = 32
BATCH = 16384

NUM_CORES = 2
NUM_SUBCORES = 16
LANES = 16
NUM_WORKERS = NUM_CORES * NUM_SUBCORES
B_PER_W = BATCH // NUM_WORKERS
GROUPS = B_PER_W // LANES


def _rsqrt_newton(x):
  y = plsc.bitcast(
      jnp.int32(0x5F3759DF) - lax.shift_right_logical(
          plsc.bitcast(x, jnp.int32), jnp.int32(1)),
      jnp.float32)
  half_x = x * 0.5
  for _ in range(3):
    y = y * (1.5 - half_x * y * y)
  return y


def _body(node_hbm, table_hbm, out_hbm, idx_v, rows_v, sem):
  wid = lax.axis_index("s") * NUM_CORES + lax.axis_index("c")
  base = wid * B_PER_W
  pltpu.sync_copy(node_hbm.at[pl.ds(base, B_PER_W)], idx_v)
  pltpu.async_copy(table_hbm.at[idx_v], rows_v, sem).wait()

  lane_iota = lax.iota(jnp.int32, LANES)
  perms = [lane_iota ^ d for d in (1, 2, 4, 8)]

  def norm_row(i, carry):
    a = rows_v[i, pl.ds(0, LANES)]
    b = rows_v[i, pl.ds(LANES, LANES)]
    s = a * a + b * b
    for p in perms:
      s = s + jnp.take_along_axis(s, p, axis=0)
    scale = _rsqrt_newton(jnp.maximum(s, 1e-24))
    rows_v[i, pl.ds(0, LANES)] = a * scale
    rows_v[i, pl.ds(LANES, LANES)] = b * scale
    return carry

  lax.fori_loop(0, B_PER_W, norm_row, 0)
  pltpu.sync_copy(rows_v, out_hbm.at[pl.ds(base, B_PER_W)])


@jax.jit
def _lookup_normalize(node, table):
  mesh = plsc.VectorSubcoreMesh(
      core_axis_name="c", subcore_axis_name="s",
      num_cores=NUM_CORES, num_subcores=NUM_SUBCORES)
  return pl.kernel(
      _body,
      out_type=jax.ShapeDtypeStruct((BATCH, EMBED_DIM), jnp.float32),
      mesh=mesh,
      scratch_types=[
          pltpu.VMEM((B_PER_W,), jnp.int32),
          pltpu.VMEM((B_PER_W, EMBED_DIM), jnp.float32),
          pltpu.SemaphoreType.DMA,
      ],
      compiler_params=pltpu.CompilerParams(
          needs_layout_passes=False, use_tc_tiling_on_sc=False),
  )(node, table)


def kernel(node, table):
  return _lookup_normalize(node.astype(jnp.int32), table)


# speedup vs baseline: 1.5447x; 1.5447x over previous
import jax
import jax.numpy as jnp
from jax import lax
from jax.experimental import pallas as pl
from jax.experimental.pallas import tpu as pltpu
from jax.experimental.pallas import tpu_sc as plsc

NODE_SIZE = 1000000
EMBED_DIM = 32
BATCH = 16384

NUM_CORES = 2
NUM_SUBCORES = 16
LANES = 16
NUM_WORKERS = NUM_CORES * NUM_SUBCORES
B_PER_W = BATCH // NUM_WORKERS
CHUNK = 16
NUM_CHUNKS = B_PER_W // CHUNK


def _rsqrt_newton(x):
  y = plsc.bitcast(
      jnp.int32(0x5F3759DF) - lax.shift_right_logical(
          plsc.bitcast(x, jnp.int32), jnp.int32(1)),
      jnp.float32)
  half_x = x * 0.5
  for _ in range(3):
    y = y * (1.5 - half_x * y * y)
  return y


def _body(node_hbm, table_hbm, out_hbm, idx_v, rows_v, sem):
  wid = lax.axis_index("s") * NUM_CORES + lax.axis_index("c")
  base = wid * B_PER_W
  pltpu.sync_copy(node_hbm.at[pl.ds(base, B_PER_W)], idx_v)

  def fetch_chunk(c, carry):
    vec = idx_v[pl.ds(c * CHUNK, CHUNK)]
    copies = []
    for j in range(CHUNK):
      r = lax.squeeze(lax.slice(vec, [j], [j + 1]), [0])
      copies.append(
          pltpu.async_copy(table_hbm.at[r], rows_v.at[c * CHUNK + j], sem))
    for cp in copies:
      cp.wait()
    return carry

  lax.fori_loop(0, NUM_CHUNKS, fetch_chunk, 0)

  lane_iota = lax.iota(jnp.int32, LANES)
  perms = [lane_iota ^ d for d in (1, 2, 4, 8)]

  def norm_row(i, carry):
    a = rows_v[i, pl.ds(0, LANES)]
    b = rows_v[i, pl.ds(LANES, LANES)]
    s = a * a + b * b
    for p in perms:
      s = s + jnp.take_along_axis(s, p, axis=0)
    scale = _rsqrt_newton(jnp.maximum(s, 1e-24))
    rows_v[i, pl.ds(0, LANES)] = a * scale
    rows_v[i, pl.ds(LANES, LANES)] = b * scale
    return carry

  lax.fori_loop(0, B_PER_W, norm_row, 0)
  pltpu.sync_copy(rows_v, out_hbm.at[pl.ds(base, B_PER_W)])


@jax.jit
def _lookup_normalize(node, table):
  mesh = plsc.VectorSubcoreMesh(
      core_axis_name="c", subcore_axis_name="s",
      num_cores=NUM_CORES, num_subcores=NUM_SUBCORES)
  return pl.kernel(
      _body,
      out_type=jax.ShapeDtypeStruct((BATCH, EMBED_DIM), jnp.float32),
      mesh=mesh,
      scratch_types=[
          pltpu.VMEM((B_PER_W,), jnp.int32),
          pltpu.VMEM((B_PER_W, EMBED_DIM), jnp.float32),
          pltpu.SemaphoreType.DMA,
      ],
      compiler_params=pltpu.CompilerParams(needs_layout_passes=False),
  )(node, table)


def kernel(node, table):
  return _lookup_normalize(node.astype(jnp.int32), table)
